# baseline (speedup 1.0000x reference)
import functools

import jax
import jax.numpy as jnp
from jax import lax
from jax.experimental import pallas as pl
from jax.experimental.pallas import tpu as pltpu

_HEADS = 4


def _onehot_t(bi_ref, B, BN):
    bi = bi_ref[0]
    rows = lax.broadcasted_iota(jnp.int32, (B, BN), 0)
    return (bi == rows).astype(jnp.float32)


def _body1(B, BN, scale,
           x_ref, bi_ref, p_ref, wq_ref, bq_ref, wk_ref, bk_ref,
           ex_ref, dnm_ref, C_s, c0_s):
    i = pl.program_id(0)

    @pl.when(i == 0)
    def _prep():
        Q = jnp.dot(p_ref[...], wq_ref[...],
                    preferred_element_type=jnp.float32) + bq_ref[...]
        C_s[...] = scale * lax.dot_general(
            Q, wk_ref[...], (((1,), (1,)), ((), ())),
            preferred_element_type=jnp.float32)
        c0_s[...] = scale * lax.dot_general(
            bk_ref[...], Q, (((1,), (1,)), ((), ())),
            preferred_element_type=jnp.float32)
        dnm_ref[...] = jnp.zeros_like(dnm_ref)

    oht = _onehot_t(bi_ref, B, BN)
    G = lax.dot_general(oht, C_s[...], (((0,), (0,)), ((), ())),
                        preferred_element_type=jnp.float32)
    c0g = lax.dot_general(c0_s[...], oht, (((1,), (0,)), ((), ())),
                          preferred_element_type=jnp.float32)
    s = jnp.sum(x_ref[...] * G, axis=1, keepdims=True)
    ex = jnp.exp(s.T + c0g)
    ex_ref[...] = ex[None]
    dnm_ref[...] += lax.dot_general(ex, oht, (((1,), (1,)), ((), ())),
                                    preferred_element_type=jnp.float32)


def _body2(B, BN,
           x_ref, bi_ref, ex_ref, dnm_ref, wv_ref, bv_ref, g_ref, b_ref,
           out_ref, attn_ref):
    oht = _onehot_t(bi_ref, B, BN)
    dg = lax.dot_general(dnm_ref[...], oht, (((1,), (0,)), ((), ())),
                         preferred_element_type=jnp.float32)
    attn = ex_ref[0] / (dg + 1e-16)
    attn_ref[...] = attn[None]
    x = x_ref[...]
    V = jnp.dot(x, wv_ref[...],
                preferred_element_type=jnp.float32) + bv_ref[...]
    out = attn.T * V + x
    mu = jnp.mean(out, axis=1, keepdims=True)
    d = out - mu
    var = jnp.mean(d * d, axis=1, keepdims=True)
    out_ref[...] = d * lax.rsqrt(var + 1e-5) * g_ref[...] + b_ref[...]


def kernel(drug_atoms, protein_ctx, batch_index, Wq, bq, Wk, bk, Wv, bv,
           ln_g, ln_b):
    N, H = drug_atoms.shape
    B = protein_ctx.shape[0]
    scale = float(H // _HEADS) ** -0.5

    BN = 2048
    ntot = -(-N // BN) * BN
    npad = ntot - N
    x = drug_atoms
    bi = batch_index
    if npad:
        x = jnp.concatenate([x, jnp.zeros((npad, H), jnp.float32)], axis=0)
        bi = jnp.concatenate([bi, jnp.full((npad,), B, jnp.int32)])
    nblk = ntot // BN
    bi3 = bi.reshape(nblk, 1, BN)

    full = lambda shape: pl.BlockSpec(shape, lambda i: tuple(0 for _ in shape))
    blocked = lambda shape: pl.BlockSpec(
        shape, lambda i: (i,) + (0,) * (len(shape) - 1))

    ex3, dnm = pl.pallas_call(
        functools.partial(_body1, B, BN, scale),
        grid=(nblk,),
        in_specs=[
            blocked((BN, H)),
            blocked((1, 1, BN)),
            full((B, H)),
            full((H, H)),
            full((1, H)),
            full((H, H)),
            full((1, H)),
        ],
        out_specs=[blocked((1, 1, BN)), full((1, B))],
        out_shape=[
            jax.ShapeDtypeStruct((nblk, 1, BN), jnp.float32),
            jax.ShapeDtypeStruct((1, B), jnp.float32),
        ],
        scratch_shapes=[
            pltpu.VMEM((B, H), jnp.float32),
            pltpu.VMEM((1, B), jnp.float32),
        ],
    )(x, bi3, protein_ctx, Wq, bq.reshape(1, H), Wk, bk.reshape(1, H))

    normed, attn3 = pl.pallas_call(
        functools.partial(_body2, B, BN),
        grid=(nblk,),
        in_specs=[
            blocked((BN, H)),
            blocked((1, 1, BN)),
            blocked((1, 1, BN)),
            full((1, B)),
            full((H, H)),
            full((1, H)),
            full((1, H)),
            full((1, H)),
        ],
        out_specs=[blocked((BN, H)), blocked((1, 1, BN))],
        out_shape=[
            jax.ShapeDtypeStruct((ntot, H), jnp.float32),
            jax.ShapeDtypeStruct((nblk, 1, BN), jnp.float32),
        ],
    )(x, bi3, ex3, dnm, Wv, bv.reshape(1, H), ln_g.reshape(1, H),
      ln_b.reshape(1, H))

    attn = attn3.reshape(ntot, 1)
    if npad:
        normed = normed[:N]
        attn = attn[:N]
    return (normed, attn)


# speedup vs baseline: 1.1220x; 1.1220x over previous
import functools

import jax
import jax.numpy as jnp
from jax import lax
from jax.experimental import pallas as pl
from jax.experimental.pallas import tpu as pltpu
from jax.experimental.pallas import tpu_sc as plsc

_HEADS = 4
_NC = 2
_NS = 16
_NW = _NC * _NS
_L = 16

_SC_PARAMS = pltpu.CompilerParams(needs_layout_passes=False)


def _take16(v, idx):
    dnums = lax.GatherDimensionNumbers(
        offset_dims=(), collapsed_slice_dims=(0,), start_index_map=(0,))
    return lax.gather(v, idx[:, None], dnums, slice_sizes=(1,),
                      mode=lax.GatherScatterMode.PROMISE_IN_BOUNDS)


def _body1(B, BN, N, scale,
           x_ref, bi_ref, p_ref, wq_ref, bq_ref, wk_ref,
           ex_ref, C_s):
    i = pl.program_id(0)

    @pl.when(i == 0)
    def _prep():
        Q = jnp.dot(p_ref[...], wq_ref[...],
                    preferred_element_type=jnp.float32) + bq_ref[...]
        C_s[...] = scale * lax.dot_general(
            Q, wk_ref[...], (((1,), (1,)), ((), ())),
            preferred_element_type=jnp.float32)

    bir = bi_ref[0]
    oht = (bir == lax.broadcasted_iota(jnp.int32, (B, BN), 0)
           ).astype(jnp.float32)
    G = lax.dot_general(oht, C_s[...], (((0,), (0,)), ((), ())),
                        preferred_element_type=jnp.float32)
    dots = jnp.sum(x_ref[...] * G, axis=1, keepdims=True)
    ex = jnp.exp(dots)
    row = i * BN + lax.broadcasted_iota(jnp.int32, (BN, 1), 0)
    ex_ref[...] = jnp.where(row < N, ex, 0.0)


def _make_sc_segsum(ntot, B, per_w):
    nv = per_w // _L
    mesh = plsc.VectorSubcoreMesh(core_axis_name="c", subcore_axis_name="s")

    @functools.partial(
        pl.kernel, mesh=mesh,
        out_type=jax.ShapeDtypeStruct((_NW * B,), jnp.float32),
        compiler_params=_SC_PARAMS,
        scratch_types=[
            pltpu.VMEM((per_w,), jnp.int32),
            pltpu.VMEM((per_w,), jnp.float32),
            pltpu.VMEM((B,), jnp.float32),
        ],
    )
    def sc_segsum(idx_hbm, ex_hbm, part_hbm, idx_v, ex_v, acc_v):
        wid = lax.axis_index("s") * _NC + lax.axis_index("c")
        base = wid * per_w

        def _zero(j, c):
            acc_v[pl.ds(j * _L, _L)] = jnp.zeros((_L,), jnp.float32)
            return c

        lax.fori_loop(0, B // _L, _zero, 0)
        pltpu.sync_copy(idx_hbm.at[pl.ds(base, per_w)], idx_v)
        pltpu.sync_copy(ex_hbm.at[pl.ds(base, per_w)], ex_v)
        lane = lax.iota(jnp.int32, _L)

        def _acc(k, c):
            sl = pl.ds(k * _L, _L)
            bi = idx_v[sl]
            ev = ex_v[sl]
            s = plsc.cumsum(ev)
            nxt = _take16(bi, jnp.minimum(lane + 1, _L - 1))
            bmask = (bi != nxt) | (lane == _L - 1)
            vb = jnp.where(bmask, s, 0.0)
            vsh = _take16(vb, jnp.maximum(lane - 1, 0))
            vsh = jnp.where(lane == 0, 0.0, vsh)
            w = plsc.cummax(vsh)
            part = s - w
            dg = plsc.load_gather(acc_v, [bi])
            plsc.store_scatter(acc_v, [bi], dg + part, mask=bmask)
            return c

        lax.fori_loop(0, nv, _acc, 0)
        pltpu.sync_copy(acc_v, part_hbm.at[pl.ds(wid * B, B)])

    return sc_segsum


def _make_sc_attn(ntot, B, per_w):
    nv = per_w // _L
    mesh = plsc.VectorSubcoreMesh(core_axis_name="c", subcore_axis_name="s")

    @functools.partial(
        pl.kernel, mesh=mesh,
        out_type=jax.ShapeDtypeStruct((ntot,), jnp.float32),
        compiler_params=_SC_PARAMS,
        scratch_types=[
            pltpu.VMEM((_NW * B,), jnp.float32),
            pltpu.VMEM((B,), jnp.float32),
            pltpu.VMEM((per_w,), jnp.int32),
            pltpu.VMEM((per_w,), jnp.float32),
            pltpu.VMEM((per_w,), jnp.float32),
        ],
    )
    def sc_attn(idx_hbm, ex_hbm, part_hbm, attn_hbm,
                part_v, dnm_v, idx_v, ex_v, at_v):
        wid = lax.axis_index("s") * _NC + lax.axis_index("c")
        base = wid * per_w
        pltpu.sync_copy(part_hbm, part_v)

        def _red(j, c):
            acc = jnp.zeros((_L,), jnp.float32)
            for t in range(_NW):
                acc = acc + part_v[pl.ds(t * B + j * _L, _L)]
            dnm_v[pl.ds(j * _L, _L)] = acc
            return c

        lax.fori_loop(0, B // _L, _red, 0)
        pltpu.sync_copy(idx_hbm.at[pl.ds(base, per_w)], idx_v)
        pltpu.sync_copy(ex_hbm.at[pl.ds(base, per_w)], ex_v)

        def _att(k, c):
            sl = pl.ds(k * _L, _L)
            bi = idx_v[sl]
            dg = plsc.load_gather(dnm_v, [bi])
            at_v[sl] = ex_v[sl] / (dg + 1e-16)
            return c

        lax.fori_loop(0, nv, _att, 0)
        pltpu.sync_copy(at_v, attn_hbm.at[pl.ds(base, per_w)])

    return sc_attn


def _body2(BN, x_ref, attn_ref, wv_ref, bv_ref, g_ref, b_ref, out_ref):
    attn = attn_ref[...]
    x = x_ref[...]
    V = jnp.dot(x, wv_ref[...],
                preferred_element_type=jnp.float32) + bv_ref[...]
    out = attn * V + x
    mu = jnp.mean(out, axis=1, keepdims=True)
    d = out - mu
    var = jnp.mean(d * d, axis=1, keepdims=True)
    out_ref[...] = d * lax.rsqrt(var + 1e-5) * g_ref[...] + b_ref[...]


def kernel(drug_atoms, protein_ctx, batch_index, Wq, bq, Wk, bk, Wv, bv,
           ln_g, ln_b):
    N, H = drug_atoms.shape
    B = protein_ctx.shape[0]
    scale = float(H // _HEADS) ** -0.5

    BN = 2048
    ntot = -(-N // BN) * BN
    npad = ntot - N
    per_w = ntot // _NW

    x = drug_atoms
    bi = batch_index
    if npad:
        x = jnp.concatenate([x, jnp.zeros((npad, H), jnp.float32)], axis=0)
        bi = jnp.concatenate([bi, jnp.zeros((npad,), jnp.int32)])
    nblk = ntot // BN
    bi3 = bi.reshape(nblk, 1, BN)

    full = lambda shape: pl.BlockSpec(shape, lambda i: tuple(0 for _ in shape))
    blocked = lambda shape: pl.BlockSpec(
        shape, lambda i: (i,) + (0,) * (len(shape) - 1))

    ex = pl.pallas_call(
        functools.partial(_body1, B, BN, N, scale),
        grid=(nblk,),
        in_specs=[
            blocked((BN, H)),
            blocked((1, 1, BN)),
            full((B, H)),
            full((H, H)),
            full((1, H)),
            full((H, H)),
        ],
        out_specs=blocked((BN, 1)),
        out_shape=jax.ShapeDtypeStruct((ntot, 1), jnp.float32),
        scratch_shapes=[pltpu.VMEM((B, H), jnp.float32)],
    )(x, bi3, protein_ctx, Wq, bq.reshape(1, H), Wk)

    ex1 = ex.reshape(ntot)
    parts = _make_sc_segsum(ntot, B, per_w)(bi, ex1)
    attn1 = _make_sc_attn(ntot, B, per_w)(bi, ex1, parts)
    attn = attn1.reshape(ntot, 1)

    normed = pl.pallas_call(
        functools.partial(_body2, BN),
        grid=(nblk,),
        in_specs=[
            blocked((BN, H)),
            blocked((BN, 1)),
            full((H, H)),
            full((1, H)),
            full((1, H)),
            full((1, H)),
        ],
        out_specs=blocked((BN, H)),
        out_shape=jax.ShapeDtypeStruct((ntot, H), jnp.float32),
    )(x, attn, Wv, bv.reshape(1, H), ln_g.reshape(1, H), ln_b.reshape(1, H))

    if npad:
        normed = normed[:N]
        attn = attn[:N]
    return (normed, attn)


# speedup vs baseline: 1.2137x; 1.0817x over previous
import functools

import jax
import jax.numpy as jnp
from jax import lax
from jax.experimental import pallas as pl
from jax.experimental.pallas import tpu as pltpu
from jax.experimental.pallas import tpu_sc as plsc

_HEADS = 4
_NC = 2
_NS = 16
_NW = _NC * _NS
_L = 16

_SC_PARAMS = pltpu.CompilerParams(needs_layout_passes=False)


def _take16(v, idx):
    dnums = lax.GatherDimensionNumbers(
        offset_dims=(), collapsed_slice_dims=(0,), start_index_map=(0,))
    return lax.gather(v, idx[:, None], dnums, slice_sizes=(1,),
                      mode=lax.GatherScatterMode.PROMISE_IN_BOUNDS)


def _body1(B, BN, N, scale,
           x_ref, bi_ref, p_ref, wq_ref, bq_ref, wk_ref,
           ex_ref, C_s):
    i = pl.program_id(0)

    @pl.when(i == 0)
    def _prep():
        Q = jnp.dot(p_ref[...], wq_ref[...],
                    preferred_element_type=jnp.float32) + bq_ref[...]
        C_s[...] = (scale * lax.dot_general(
            Q, wk_ref[...], (((1,), (1,)), ((), ())),
            preferred_element_type=jnp.float32)).astype(jnp.bfloat16)

    bir = bi_ref[0]
    oht = (bir == lax.broadcasted_iota(jnp.int32, (B, BN), 0)
           ).astype(jnp.bfloat16)
    G = lax.dot_general(oht, C_s[...], (((0,), (0,)), ((), ())),
                        preferred_element_type=jnp.float32)
    dots = jnp.sum(x_ref[...] * G, axis=1, keepdims=True)
    ex = jnp.exp(dots)
    row = i * BN + lax.broadcasted_iota(jnp.int32, (BN, 1), 0)
    ex_ref[...] = jnp.where(row < N, ex, 0.0)


def _make_sc_softmax(ntot, B):
    per_t = ntot // _NS
    per_w = ntot // _NW
    nv = per_t // _L
    nva = per_w // _L
    mesh = plsc.VectorSubcoreMesh(core_axis_name="c", subcore_axis_name="s")

    @functools.partial(
        pl.kernel, mesh=mesh,
        out_type=jax.ShapeDtypeStruct((ntot,), jnp.float32),
        compiler_params=_SC_PARAMS,
        scratch_types=[
            pltpu.VMEM((per_t,), jnp.int32),
            pltpu.VMEM((per_t,), jnp.float32),
            pltpu.VMEM((B,), jnp.float32),
            pltpu.VMEM((_NS * B,), jnp.float32),
            pltpu.VMEM((B,), jnp.float32),
            pltpu.VMEM((per_w,), jnp.float32),
            pltpu.VMEM_SHARED((_NS * B,), jnp.float32),
        ],
    )
    def sc_softmax(idx_hbm, ex_hbm, attn_hbm,
                   idx_v, ex_v, acc_v, part_v, dnm_v, at_v, shared):
        c = lax.axis_index("c")
        s = lax.axis_index("s")
        tbase = s * per_t

        def _zero(j, k):
            acc_v[pl.ds(j * _L, _L)] = jnp.zeros((_L,), jnp.float32)
            return k

        lax.fori_loop(0, B // _L, _zero, 0)
        pltpu.sync_copy(idx_hbm.at[pl.ds(tbase, per_t)], idx_v)
        pltpu.sync_copy(ex_hbm.at[pl.ds(tbase, per_t)], ex_v)
        lane = lax.iota(jnp.int32, _L)

        def _acc(k, cr):
            sl = pl.ds(k * _L, _L)
            bi = idx_v[sl]
            ev = ex_v[sl]
            sm = plsc.cumsum(ev)
            nxt = _take16(bi, jnp.minimum(lane + 1, _L - 1))
            bmask = (bi != nxt) | (lane == _L - 1)
            vb = jnp.where(bmask, sm, 0.0)
            vsh = _take16(vb, jnp.maximum(lane - 1, 0))
            vsh = jnp.where(lane == 0, 0.0, vsh)
            w = plsc.cummax(vsh)
            part = sm - w
            dg = plsc.load_gather(acc_v, [bi])
            plsc.store_scatter(acc_v, [bi], dg + part, mask=bmask)
            return cr

        lax.fori_loop(0, nv, _acc, 0)
        pltpu.sync_copy(acc_v, shared.at[pl.ds(s * B, B)])
        plsc.subcore_barrier()
        pltpu.sync_copy(shared, part_v)

        def _red(j, k):
            acc = jnp.zeros((_L,), jnp.float32)
            for t in range(_NS):
                acc = acc + part_v[pl.ds(t * B + j * _L, _L)]
            dnm_v[pl.ds(j * _L, _L)] = acc
            return k

        lax.fori_loop(0, B // _L, _red, 0)
        loc = c * per_w

        def _att(k, cr):
            sl = pl.ds(loc + k * _L, _L)
            bi = idx_v[sl]
            dg = plsc.load_gather(dnm_v, [bi])
            at_v[pl.ds(k * _L, _L)] = ex_v[sl] / (dg + 1e-16)
            return cr

        lax.fori_loop(0, nva, _att, 0)
        pltpu.sync_copy(at_v, attn_hbm.at[pl.ds(tbase + loc, per_w)])

    return sc_softmax


def _body2(BN, x_ref, attn_ref, wv_ref, bv_ref, g_ref, b_ref, out_ref):
    attn = attn_ref[...]
    x = x_ref[...]
    V = jnp.dot(x, wv_ref[...],
                preferred_element_type=jnp.float32) + bv_ref[...]
    out = attn * V + x
    mu = jnp.mean(out, axis=1, keepdims=True)
    d = out - mu
    var = jnp.mean(d * d, axis=1, keepdims=True)
    out_ref[...] = d * lax.rsqrt(var + 1e-5) * g_ref[...] + b_ref[...]


def kernel(drug_atoms, protein_ctx, batch_index, Wq, bq, Wk, bk, Wv, bv,
           ln_g, ln_b):
    N, H = drug_atoms.shape
    B = protein_ctx.shape[0]
    scale = float(H // _HEADS) ** -0.5

    BN = 2048
    ntot = -(-N // BN) * BN
    npad = ntot - N

    x = drug_atoms
    bi = batch_index
    if npad:
        x = jnp.concatenate([x, jnp.zeros((npad, H), jnp.float32)], axis=0)
        bi = jnp.concatenate([bi, jnp.zeros((npad,), jnp.int32)])
    nblk = ntot // BN
    bi3 = bi.reshape(nblk, 1, BN)

    full = lambda shape: pl.BlockSpec(shape, lambda i: tuple(0 for _ in shape))
    blocked = lambda shape: pl.BlockSpec(
        shape, lambda i: (i,) + (0,) * (len(shape) - 1))

    ex = pl.pallas_call(
        functools.partial(_body1, B, BN, N, scale),
        grid=(nblk,),
        in_specs=[
            blocked((BN, H)),
            blocked((1, 1, BN)),
            full((B, H)),
            full((H, H)),
            full((1, H)),
            full((H, H)),
        ],
        out_specs=blocked((BN, 1)),
        out_shape=jax.ShapeDtypeStruct((ntot, 1), jnp.float32),
        scratch_shapes=[pltpu.VMEM((B, H), jnp.bfloat16)],
    )(x, bi3, protein_ctx, Wq, bq.reshape(1, H), Wk)

    ex1 = ex.reshape(ntot)
    attn1 = _make_sc_softmax(ntot, B)(bi, ex1)
    attn = attn1.reshape(ntot, 1)

    normed = pl.pallas_call(
        functools.partial(_body2, BN),
        grid=(nblk,),
        in_specs=[
            blocked((BN, H)),
            blocked((BN, 1)),
            full((H, H)),
            full((1, H)),
            full((1, H)),
            full((1, H)),
        ],
        out_specs=blocked((BN, H)),
        out_shape=jax.ShapeDtypeStruct((ntot, H), jnp.float32),
    )(x, attn, Wv, bv.reshape(1, H), ln_g.reshape(1, H), ln_b.reshape(1, H))

    if npad:
        normed = normed[:N]
        attn = attn[:N]
    return (normed, attn)


# speedup vs baseline: 1.2960x; 1.0678x over previous
import functools

import jax
import jax.numpy as jnp
from jax import lax
from jax.experimental import pallas as pl
from jax.experimental.pallas import tpu as pltpu
from jax.experimental.pallas import tpu_sc as plsc

_HEADS = 4
_NC = 2
_NS = 16
_NW = _NC * _NS
_L = 16

_SC_PARAMS = pltpu.CompilerParams(needs_layout_passes=False)


def _take16(v, idx):
    dnums = lax.GatherDimensionNumbers(
        offset_dims=(), collapsed_slice_dims=(0,), start_index_map=(0,))
    return lax.gather(v, idx[:, None], dnums, slice_sizes=(1,),
                      mode=lax.GatherScatterMode.PROMISE_IN_BOUNDS)


def _body1(B, BN, N, scale,
           x_ref, bi_ref, p_ref, wq_ref, bq_ref, wk_ref,
           ex_ref, C_s):
    i = pl.program_id(0)

    @pl.when(i == 0)
    def _prep():
        Q = jnp.dot(p_ref[...], wq_ref[...],
                    preferred_element_type=jnp.float32) + bq_ref[...]
        C_s[...] = (scale * lax.dot_general(
            Q, wk_ref[...], (((1,), (1,)), ((), ())),
            preferred_element_type=jnp.float32)).astype(jnp.bfloat16)

    bir = bi_ref[0]
    oht = (bir == lax.broadcasted_iota(jnp.int32, (B, BN), 0)
           ).astype(jnp.bfloat16)
    G = lax.dot_general(oht, C_s[...], (((0,), (0,)), ((), ())),
                        preferred_element_type=jnp.float32)
    dots = jnp.sum(x_ref[...] * G, axis=1, keepdims=True)
    ex = jnp.exp(dots)
    row = i * BN + lax.broadcasted_iota(jnp.int32, (BN, 1), 0)
    ex_ref[...] = jnp.where(row < N, ex, 0.0)


def _make_sc_softmax(ntot, B):
    per_t = ntot // _NS
    per_w = ntot // _NW
    nv = per_t // _L
    nva = per_w // _L
    mesh = plsc.VectorSubcoreMesh(core_axis_name="c", subcore_axis_name="s")

    @functools.partial(
        pl.kernel, mesh=mesh,
        out_type=jax.ShapeDtypeStruct((ntot,), jnp.float32),
        compiler_params=_SC_PARAMS,
        scratch_types=[
            pltpu.VMEM((per_t,), jnp.int32),
            pltpu.VMEM((per_t,), jnp.float32),
            pltpu.VMEM((B,), jnp.float32),
            pltpu.VMEM((_NS * B,), jnp.float32),
            pltpu.VMEM((B,), jnp.float32),
            pltpu.VMEM((per_w,), jnp.float32),
            pltpu.VMEM_SHARED((_NS * B,), jnp.float32),
        ],
    )
    def sc_softmax(idx_hbm, ex_hbm, attn_hbm,
                   idx_v, ex_v, acc_v, part_v, dnm_v, at_v, shared):
        c = lax.axis_index("c")
        s = lax.axis_index("s")
        tbase = s * per_t

        def _zero(j, k):
            acc_v[pl.ds(j * _L, _L)] = jnp.zeros((_L,), jnp.float32)
            return k

        lax.fori_loop(0, B // _L, _zero, 0)
        pltpu.sync_copy(idx_hbm.at[pl.ds(tbase, per_t)], idx_v)
        pltpu.sync_copy(ex_hbm.at[pl.ds(tbase, per_t)], ex_v)
        lane = lax.iota(jnp.int32, _L)

        def _acc(k, cr):
            sl = pl.ds(k * _L, _L)
            bi = idx_v[sl]
            ev = ex_v[sl]
            sm = plsc.cumsum(ev)
            nxt = _take16(bi, jnp.minimum(lane + 1, _L - 1))
            bmask = (bi != nxt) | (lane == _L - 1)
            vb = jnp.where(bmask, sm, 0.0)
            vsh = _take16(vb, jnp.maximum(lane - 1, 0))
            vsh = jnp.where(lane == 0, 0.0, vsh)
            w = plsc.cummax(vsh)
            part = sm - w
            dg = plsc.load_gather(acc_v, [bi])
            plsc.store_scatter(acc_v, [bi], dg + part, mask=bmask)
            return cr

        lax.fori_loop(0, nv, _acc, 0)
        pltpu.sync_copy(acc_v, shared.at[pl.ds(s * B, B)])
        plsc.subcore_barrier()
        pltpu.sync_copy(shared, part_v)

        def _red(j, k):
            acc = jnp.zeros((_L,), jnp.float32)
            for t in range(_NS):
                acc = acc + part_v[pl.ds(t * B + j * _L, _L)]
            dnm_v[pl.ds(j * _L, _L)] = acc
            return k

        lax.fori_loop(0, B // _L, _red, 0)
        loc = c * per_w

        def _att(k, cr):
            sl = pl.ds(loc + k * _L, _L)
            bi = idx_v[sl]
            dg = plsc.load_gather(dnm_v, [bi])
            at_v[pl.ds(k * _L, _L)] = ex_v[sl] / (dg + 1e-16)
            return cr

        lax.fori_loop(0, nva, _att, 0)
        pltpu.sync_copy(at_v, attn_hbm.at[pl.ds(tbase + loc, per_w)])

    return sc_softmax


def _body2(BN, x_ref, attn_ref, wv_ref, bv_ref, g_ref, b_ref, out_ref):
    attn = attn_ref[...]
    x = x_ref[...]
    V = jnp.dot(x, wv_ref[...],
                preferred_element_type=jnp.float32) + bv_ref[...]
    out = attn * V + x
    mu = jnp.mean(out, axis=1, keepdims=True)
    d = out - mu
    var = jnp.mean(d * d, axis=1, keepdims=True)
    out_ref[...] = d * lax.rsqrt(var + 1e-5) * g_ref[...] + b_ref[...]


def kernel(drug_atoms, protein_ctx, batch_index, Wq, bq, Wk, bk, Wv, bv,
           ln_g, ln_b):
    N, H = drug_atoms.shape
    B = protein_ctx.shape[0]
    scale = float(H // _HEADS) ** -0.5

    BN = 4096
    ntot = -(-N // BN) * BN
    npad = ntot - N

    x = drug_atoms
    bi = batch_index
    if npad:
        x = jnp.concatenate([x, jnp.zeros((npad, H), jnp.float32)], axis=0)
        bi = jnp.concatenate([bi, jnp.zeros((npad,), jnp.int32)])
    nblk = ntot // BN
    bi3 = bi.reshape(nblk, 1, BN)

    full = lambda shape: pl.BlockSpec(shape, lambda i: tuple(0 for _ in shape))
    blocked = lambda shape: pl.BlockSpec(
        shape, lambda i: (i,) + (0,) * (len(shape) - 1))

    ex = pl.pallas_call(
        functools.partial(_body1, B, BN, N, scale),
        grid=(nblk,),
        in_specs=[
            blocked((BN, H)),
            blocked((1, 1, BN)),
            full((B, H)),
            full((H, H)),
            full((1, H)),
            full((H, H)),
        ],
        out_specs=blocked((BN, 1)),
        out_shape=jax.ShapeDtypeStruct((ntot, 1), jnp.float32),
        scratch_shapes=[pltpu.VMEM((B, H), jnp.bfloat16)],
    )(x, bi3, protein_ctx, Wq, bq.reshape(1, H), Wk)

    ex1 = ex.reshape(ntot)
    attn1 = _make_sc_softmax(ntot, B)(bi, ex1)
    attn = attn1.reshape(ntot, 1)

    normed = pl.pallas_call(
        functools.partial(_body2, BN),
        grid=(nblk,),
        in_specs=[
            blocked((BN, H)),
            blocked((BN, 1)),
            full((H, H)),
            full((1, H)),
            full((1, H)),
            full((1, H)),
        ],
        out_specs=blocked((BN, H)),
        out_shape=jax.ShapeDtypeStruct((ntot, H), jnp.float32),
    )(x, attn, Wv, bv.reshape(1, H), ln_g.reshape(1, H), ln_b.reshape(1, H))

    if npad:
        normed = normed[:N]
        attn = attn[:N]
    return (normed, attn)


# speedup vs baseline: 1.3198x; 1.0184x over previous
import functools

import jax
import jax.numpy as jnp
from jax import lax
from jax.experimental import pallas as pl
from jax.experimental.pallas import tpu as pltpu
from jax.experimental.pallas import tpu_sc as plsc

_HEADS = 4
_NC = 2
_NS = 16
_NW = _NC * _NS
_L = 16

_SC_PARAMS = pltpu.CompilerParams(needs_layout_passes=False)


def _take16(v, idx):
    dnums = lax.GatherDimensionNumbers(
        offset_dims=(), collapsed_slice_dims=(0,), start_index_map=(0,))
    return lax.gather(v, idx[:, None], dnums, slice_sizes=(1,),
                      mode=lax.GatherScatterMode.PROMISE_IN_BOUNDS)


def _body1(B, BN, N, scale,
           x_ref, bi_ref, p_ref, wq_ref, bq_ref, wk_ref,
           ex_ref, C_s):
    i = pl.program_id(0)

    @pl.when(i == 0)
    def _prep():
        Q = jnp.dot(p_ref[...], wq_ref[...],
                    preferred_element_type=jnp.float32) + bq_ref[...]
        C_s[...] = (scale * lax.dot_general(
            Q, wk_ref[...], (((1,), (1,)), ((), ())),
            preferred_element_type=jnp.float32)).astype(jnp.bfloat16)

    bir = bi_ref[0]
    oht = (bir == lax.broadcasted_iota(jnp.int32, (B, BN), 0)
           ).astype(jnp.bfloat16)
    G = lax.dot_general(oht, C_s[...], (((0,), (0,)), ((), ())),
                        preferred_element_type=jnp.float32)
    dots = jnp.sum(x_ref[...] * G, axis=1, keepdims=True)
    ex = jnp.exp(dots)
    row = i * BN + lax.broadcasted_iota(jnp.int32, (BN, 1), 0)
    ex_ref[...] = jnp.where(row < N, ex, 0.0)


def _make_sc_softmax(ntot, B):
    per_t = ntot // _NS
    per_w = ntot // _NW
    nv = per_t // _L
    nva = per_w // _L
    mesh = plsc.VectorSubcoreMesh(core_axis_name="c", subcore_axis_name="s")

    @functools.partial(
        pl.kernel, mesh=mesh,
        out_type=jax.ShapeDtypeStruct((ntot,), jnp.float32),
        compiler_params=_SC_PARAMS,
        scratch_types=[
            pltpu.VMEM((per_t,), jnp.int32),
            pltpu.VMEM((per_t,), jnp.float32),
            pltpu.VMEM((B,), jnp.float32),
            pltpu.VMEM((_NS * B,), jnp.float32),
            pltpu.VMEM((B,), jnp.float32),
            pltpu.VMEM((per_w,), jnp.float32),
            pltpu.VMEM_SHARED((_NS * B,), jnp.float32),
        ],
    )
    def sc_softmax(idx_hbm, ex_hbm, attn_hbm,
                   idx_v, ex_v, acc_v, part_v, dnm_v, at_v, shared):
        c = lax.axis_index("c")
        s = lax.axis_index("s")
        tbase = s * per_t

        def _zero(j, k):
            acc_v[pl.ds(j * _L, _L)] = jnp.zeros((_L,), jnp.float32)
            return k

        lax.fori_loop(0, B // _L, _zero, 0)
        pltpu.sync_copy(idx_hbm.at[pl.ds(tbase, per_t)], idx_v)
        pltpu.sync_copy(ex_hbm.at[pl.ds(tbase, per_t)], ex_v)
        lane = lax.iota(jnp.int32, _L)

        def _acc(k, cr):
            sl = pl.ds(k * _L, _L)
            bi = idx_v[sl]
            ev = ex_v[sl]
            sm = plsc.cumsum(ev)
            nxt = _take16(bi, jnp.minimum(lane + 1, _L - 1))
            bmask = (bi != nxt) | (lane == _L - 1)
            vb = jnp.where(bmask, sm, 0.0)
            vsh = _take16(vb, jnp.maximum(lane - 1, 0))
            vsh = jnp.where(lane == 0, 0.0, vsh)
            w = plsc.cummax(vsh)
            part = sm - w
            dg = plsc.load_gather(acc_v, [bi])
            plsc.store_scatter(acc_v, [bi], dg + part, mask=bmask)
            return cr

        lax.fori_loop(0, nv, _acc, 0)
        pltpu.sync_copy(acc_v, shared.at[pl.ds(s * B, B)])
        plsc.subcore_barrier()
        pltpu.sync_copy(shared, part_v)

        def _red(j, k):
            acc = jnp.zeros((_L,), jnp.float32)
            for t in range(_NS):
                acc = acc + part_v[pl.ds(t * B + j * _L, _L)]
            dnm_v[pl.ds(j * _L, _L)] = acc
            return k

        lax.fori_loop(0, B // _L, _red, 0)
        loc = c * per_w

        def _att(k, cr):
            sl = pl.ds(loc + k * _L, _L)
            bi = idx_v[sl]
            dg = plsc.load_gather(dnm_v, [bi])
            at_v[pl.ds(k * _L, _L)] = ex_v[sl] / (dg + 1e-16)
            return cr

        lax.fori_loop(0, nva, _att, 0)
        pltpu.sync_copy(at_v, attn_hbm.at[pl.ds(tbase + loc, per_w)])

    return sc_softmax


def _body2(BN, x_ref, attn_ref, wv_ref, bv_ref, g_ref, b_ref, out_ref):
    attn = attn_ref[...]
    x = x_ref[...]
    V = jnp.dot(x, wv_ref[...],
                preferred_element_type=jnp.float32) + bv_ref[...]
    out = attn * V + x
    mu = jnp.mean(out, axis=1, keepdims=True)
    d = out - mu
    var = jnp.mean(d * d, axis=1, keepdims=True)
    out_ref[...] = d * lax.rsqrt(var + 1e-5) * g_ref[...] + b_ref[...]


def kernel(drug_atoms, protein_ctx, batch_index, Wq, bq, Wk, bk, Wv, bv,
           ln_g, ln_b):
    N, H = drug_atoms.shape
    B = protein_ctx.shape[0]
    scale = float(H // _HEADS) ** -0.5

    BN = 6400
    ntot = -(-N // BN) * BN
    npad = ntot - N

    x = drug_atoms
    bi = batch_index
    if npad:
        x = jnp.concatenate([x, jnp.zeros((npad, H), jnp.float32)], axis=0)
        bi = jnp.concatenate([bi, jnp.zeros((npad,), jnp.int32)])
    nblk = ntot // BN
    bi3 = bi.reshape(nblk, 1, BN)

    full = lambda shape: pl.BlockSpec(shape, lambda i: tuple(0 for _ in shape))
    blocked = lambda shape: pl.BlockSpec(
        shape, lambda i: (i,) + (0,) * (len(shape) - 1))

    ex = pl.pallas_call(
        functools.partial(_body1, B, BN, N, scale),
        grid=(nblk,),
        in_specs=[
            blocked((BN, H)),
            blocked((1, 1, BN)),
            full((B, H)),
            full((H, H)),
            full((1, H)),
            full((H, H)),
        ],
        out_specs=blocked((BN, 1)),
        out_shape=jax.ShapeDtypeStruct((ntot, 1), jnp.float32),
        scratch_shapes=[pltpu.VMEM((B, H), jnp.bfloat16)],
    )(x, bi3, protein_ctx, Wq, bq.reshape(1, H), Wk)

    ex1 = ex.reshape(ntot)
    attn1 = _make_sc_softmax(ntot, B)(bi, ex1)
    attn = attn1.reshape(ntot, 1)

    normed = pl.pallas_call(
        functools.partial(_body2, BN),
        grid=(nblk,),
        in_specs=[
            blocked((BN, H)),
            blocked((BN, 1)),
            full((H, H)),
            full((1, H)),
            full((1, H)),
            full((1, H)),
        ],
        out_specs=blocked((BN, H)),
        out_shape=jax.ShapeDtypeStruct((ntot, H), jnp.float32),
    )(x, attn, Wv, bv.reshape(1, H), ln_g.reshape(1, H), ln_b.reshape(1, H))

    if npad:
        normed = normed[:N]
        attn = attn[:N]
    return (normed, attn)


# speedup vs baseline: 1.3268x; 1.0054x over previous
import functools

import jax
import jax.numpy as jnp
from jax import lax
from jax.experimental import pallas as pl
from jax.experimental.pallas import tpu as pltpu
from jax.experimental.pallas import tpu_sc as plsc

_HEADS = 4
_NC = 2
_NS = 16
_NW = _NC * _NS
_L = 16

_SC_PARAMS = pltpu.CompilerParams(needs_layout_passes=False)


def _take16(v, idx):
    dnums = lax.GatherDimensionNumbers(
        offset_dims=(), collapsed_slice_dims=(0,), start_index_map=(0,))
    return lax.gather(v, idx[:, None], dnums, slice_sizes=(1,),
                      mode=lax.GatherScatterMode.PROMISE_IN_BOUNDS)


def _body1(B, BN, N, scale,
           x_ref, bi_ref, p_ref, wq_ref, bq_ref, wk_ref,
           ex_ref, C_s):
    i = pl.program_id(0)

    @pl.when(i == 0)
    def _prep():
        Q = jnp.dot(p_ref[...], wq_ref[...],
                    preferred_element_type=jnp.float32) + bq_ref[...]
        C_s[...] = (scale * lax.dot_general(
            Q, wk_ref[...], (((1,), (1,)), ((), ())),
            preferred_element_type=jnp.float32)).astype(jnp.bfloat16)

    bir = bi_ref[0]
    oht = (bir == lax.broadcasted_iota(jnp.int32, (B, BN), 0)
           ).astype(jnp.bfloat16)
    G = lax.dot_general(oht, C_s[...], (((0,), (0,)), ((), ())),
                        preferred_element_type=jnp.float32)
    dots = jnp.sum(x_ref[...] * G, axis=1, keepdims=True)
    ex = jnp.exp(dots)
    row = i * BN + lax.broadcasted_iota(jnp.int32, (BN, 1), 0)
    ex_ref[...] = jnp.where(row < N, ex, 0.0)


def _make_sc_softmax(ntot, B):
    per_t = ntot // _NS
    per_w = ntot // _NW
    nv = per_t // _L
    nva = per_w // _L
    mesh = plsc.VectorSubcoreMesh(core_axis_name="c", subcore_axis_name="s")

    @functools.partial(
        pl.kernel, mesh=mesh,
        out_type=jax.ShapeDtypeStruct((ntot,), jnp.float32),
        compiler_params=_SC_PARAMS,
        scratch_types=[
            pltpu.VMEM((per_t,), jnp.int32),
            pltpu.VMEM((per_t,), jnp.float32),
            pltpu.VMEM((B,), jnp.float32),
            pltpu.VMEM((_NS * B,), jnp.float32),
            pltpu.VMEM((B,), jnp.float32),
            pltpu.VMEM((per_w,), jnp.float32),
            pltpu.VMEM_SHARED((_NS * B,), jnp.float32),
        ],
    )
    def sc_softmax(idx_hbm, ex_hbm, attn_hbm,
                   idx_v, ex_v, acc_v, part_v, dnm_v, at_v, shared):
        c = lax.axis_index("c")
        s = lax.axis_index("s")
        tbase = s * per_t

        def _zero(j, k):
            acc_v[pl.ds(j * _L, _L)] = jnp.zeros((_L,), jnp.float32)
            return k

        lax.fori_loop(0, B // _L, _zero, 0)
        pltpu.sync_copy(idx_hbm.at[pl.ds(tbase, per_t)], idx_v)
        pltpu.sync_copy(ex_hbm.at[pl.ds(tbase, per_t)], ex_v)
        lane = lax.iota(jnp.int32, _L)

        def _acc(k, cr):
            sl = pl.ds(k * _L, _L)
            bi = idx_v[sl]
            ev = ex_v[sl]
            sm = plsc.cumsum(ev)
            nxt = _take16(bi, jnp.minimum(lane + 1, _L - 1))
            bmask = (bi != nxt) | (lane == _L - 1)
            vb = jnp.where(bmask, sm, 0.0)
            vsh = _take16(vb, jnp.maximum(lane - 1, 0))
            vsh = jnp.where(lane == 0, 0.0, vsh)
            w = plsc.cummax(vsh)
            part = sm - w
            dg = plsc.load_gather(acc_v, [bi])
            plsc.store_scatter(acc_v, [bi], dg + part, mask=bmask)
            return cr

        lax.fori_loop(0, nv, _acc, 0)
        pltpu.sync_copy(acc_v, shared.at[pl.ds(s * B, B)])
        plsc.subcore_barrier()
        pltpu.sync_copy(shared, part_v)

        def _red(j, k):
            acc = jnp.zeros((_L,), jnp.float32)
            for t in range(_NS):
                acc = acc + part_v[pl.ds(t * B + j * _L, _L)]
            dnm_v[pl.ds(j * _L, _L)] = acc
            return k

        lax.fori_loop(0, B // _L, _red, 0)
        loc = c * per_w

        def _att(k, cr):
            sl = pl.ds(loc + k * _L, _L)
            bi = idx_v[sl]
            dg = plsc.load_gather(dnm_v, [bi])
            at_v[pl.ds(k * _L, _L)] = ex_v[sl] / (dg + 1e-16)
            return cr

        lax.fori_loop(0, nva, _att, 0)
        pltpu.sync_copy(at_v, attn_hbm.at[pl.ds(tbase + loc, per_w)])

    return sc_softmax


def _body2(BN, x_ref, attn_ref, wv_ref, bv_ref, g_ref, b_ref, out_ref):
    attn = attn_ref[...]
    x = x_ref[...]
    V = jnp.dot(x, wv_ref[...],
                preferred_element_type=jnp.float32) + bv_ref[...]
    out = attn * V + x
    mu = jnp.mean(out, axis=1, keepdims=True)
    d = out - mu
    var = jnp.mean(d * d, axis=1, keepdims=True)
    out_ref[...] = d * lax.rsqrt(var + 1e-5) * g_ref[...] + b_ref[...]


def kernel(drug_atoms, protein_ctx, batch_index, Wq, bq, Wk, bk, Wv, bv,
           ln_g, ln_b):
    N, H = drug_atoms.shape
    B = protein_ctx.shape[0]
    scale = float(H // _HEADS) ** -0.5

    BN = 12800
    ntot = -(-N // BN) * BN
    npad = ntot - N

    x = drug_atoms
    bi = batch_index
    if npad:
        x = jnp.concatenate([x, jnp.zeros((npad, H), jnp.float32)], axis=0)
        bi = jnp.concatenate([bi, jnp.zeros((npad,), jnp.int32)])
    nblk = ntot // BN
    bi3 = bi.reshape(nblk, 1, BN)

    full = lambda shape: pl.BlockSpec(shape, lambda i: tuple(0 for _ in shape))
    blocked = lambda shape: pl.BlockSpec(
        shape, lambda i: (i,) + (0,) * (len(shape) - 1))

    ex = pl.pallas_call(
        functools.partial(_body1, B, BN, N, scale),
        grid=(nblk,),
        in_specs=[
            blocked((BN, H)),
            blocked((1, 1, BN)),
            full((B, H)),
            full((H, H)),
            full((1, H)),
            full((H, H)),
        ],
        out_specs=blocked((BN, 1)),
        out_shape=jax.ShapeDtypeStruct((ntot, 1), jnp.float32),
        scratch_shapes=[pltpu.VMEM((B, H), jnp.bfloat16)],
    )(x, bi3, protein_ctx, Wq, bq.reshape(1, H), Wk)

    ex1 = ex.reshape(ntot)
    attn1 = _make_sc_softmax(ntot, B)(bi, ex1)
    attn = attn1.reshape(ntot, 1)

    normed = pl.pallas_call(
        functools.partial(_body2, BN),
        grid=(nblk,),
        in_specs=[
            blocked((BN, H)),
            blocked((BN, 1)),
            full((H, H)),
            full((1, H)),
            full((1, H)),
            full((1, H)),
        ],
        out_specs=blocked((BN, H)),
        out_shape=jax.ShapeDtypeStruct((ntot, H), jnp.float32),
    )(x, attn, Wv, bv.reshape(1, H), ln_g.reshape(1, H), ln_b.reshape(1, H))

    if npad:
        normed = normed[:N]
        attn = attn[:N]
    return (normed, attn)
